# Initial kernel scaffold; baseline (speedup 1.0000x reference)
import jax
import jax.numpy as jnp
from jax.experimental import pallas as pl


def kernel(node_feats, edge_feats, distances, edge_index, mlp_W1, mlp_b1, mlp_W2, mlp_b2, lin1_W, lin1_b, lin2_W, lin2_b, lin_W, lin_b):
    raise NotImplementedError("write your pallas kernel here")


# speedup vs baseline: 1.4375x; 1.4375x over previous
import functools
import math

import jax
import jax.numpy as jnp
from jax.experimental import pallas as pl
from jax.experimental.pallas import tpu as pltpu
from jax.experimental.pallas import tpu_sc as plsc

N, E, H, D = 10000, 320000, 128, 16
CUTOFF = 10.0
_LOG2 = math.log(2.0)

NC, NS, L = 2, 16, 16
NW = NC * NS
CH = 128
NCHUNK = E // CH
BLK_ROWS = 80
NBLK = N // BLK_ROWS

EBLK = 2000


def _ssp(x):
    return jax.nn.softplus(x) - _LOG2


def _ef_body(e_ref, d_ref, w1_ref, b1_ref, w2_ref, b2_ref, out_ref):
    t = jnp.dot(e_ref[...], w1_ref[...], preferred_element_type=jnp.float32)
    t = _ssp(t + b1_ref[...])
    t = jnp.dot(t, w2_ref[...], preferred_element_type=jnp.float32) + b2_ref[...]
    c = 0.5 * (jnp.cos(d_ref[...] * (math.pi / CUTOFF)) + 1.0)
    out_ref[...] = t * c


def _edge_mlp(edge_feats, dists2d, w1t, b1, w2t, b2):
    grid = E // EBLK
    return pl.pallas_call(
        _ef_body,
        grid=(grid,),
        in_specs=[
            pl.BlockSpec((EBLK, D), lambda i: (i, 0)),
            pl.BlockSpec((EBLK, 1), lambda i: (i, 0)),
            pl.BlockSpec((D, H), lambda i: (0, 0)),
            pl.BlockSpec((1, H), lambda i: (0, 0)),
            pl.BlockSpec((H, H), lambda i: (0, 0)),
            pl.BlockSpec((1, H), lambda i: (0, 0)),
        ],
        out_specs=pl.BlockSpec((EBLK, H), lambda i: (i, 0)),
        out_shape=jax.ShapeDtypeStruct((E, H), jnp.float32),
    )(edge_feats, dists2d, w1t, b1, w2t, b2)


def _h_body(x_ref, w_ref, b_ref, out_ref):
    out_ref[...] = (
        jnp.dot(x_ref[...], w_ref[...], preferred_element_type=jnp.float32)
        + b_ref[...]
    )


def _node_lin(node_feats, wt, b):
    return pl.pallas_call(
        _h_body,
        out_shape=jax.ShapeDtypeStruct((N, H), jnp.float32),
    )(node_feats, wt, b)


def _sc_msg_agg(h, src, dst, ef):
    mesh = plsc.VectorSubcoreMesh(core_axis_name="c", subcore_axis_name="s")

    @functools.partial(
        pl.kernel,
        out_type=jax.ShapeDtypeStruct((NC, N, H), jnp.float32),
        mesh=mesh,
        scratch_types=[
            pltpu.VMEM((CH,), jnp.int32),
            pltpu.VMEM((CH,), jnp.int32),
            pltpu.VMEM((CH, H), jnp.float32),
            pltpu.VMEM((CH, H), jnp.float32),
            pltpu.VMEM((BLK_ROWS, H), jnp.float32),
            pltpu.VMEM_SHARED((N, H), jnp.float32),
            pltpu.SemaphoreType.DMA,
        ],
    )
    def k(h_hbm, src_hbm, dst_hbm, ef_hbm, out_hbm,
          src_v, dst_v, ef_v, rows_v, zero_v, acc_sh, sem):
        cid = jax.lax.axis_index("c")
        sid = jax.lax.axis_index("s")
        w = cid * NS + sid

        @pl.loop(0, BLK_ROWS)
        def _(i):
            @pl.loop(0, H, step=L)
            def _(kk):
                zero_v[i, pl.ds(kk, L)] = jnp.zeros((L,), jnp.float32)

        nblk_t = (NBLK - sid + NS - 1) // NS

        @pl.loop(0, nblk_t)
        def _(j):
            row0 = (sid + j * NS) * BLK_ROWS
            pltpu.sync_copy(zero_v, acc_sh.at[pl.ds(row0, BLK_ROWS)])

        plsc.subcore_barrier()

        nch = (NCHUNK - w + NW - 1) // NW

        @pl.loop(0, nch)
        def _(j):
            base = (w + j * NW) * CH
            pltpu.sync_copy(src_hbm.at[pl.ds(base, CH)], src_v)
            pltpu.sync_copy(dst_hbm.at[pl.ds(base, CH)], dst_v)
            pltpu.async_copy(h_hbm.at[src_v], rows_v, sem).wait()
            pltpu.sync_copy(ef_hbm.at[pl.ds(base, CH)], ef_v)

            @pl.loop(0, CH)
            def _(i):
                @pl.loop(0, H, step=L)
                def _(kk):
                    rows_v[i, pl.ds(kk, L)] = (
                        rows_v[i, pl.ds(kk, L)] * ef_v[i, pl.ds(kk, L)]
                    )

            pltpu.sync_copy(rows_v, acc_sh.at[dst_v], add=True)

        plsc.subcore_barrier()

        @pl.loop(0, nblk_t)
        def _(j):
            row0 = (sid + j * NS) * BLK_ROWS
            pltpu.sync_copy(
                acc_sh.at[pl.ds(row0, BLK_ROWS)],
                out_hbm.at[cid, pl.ds(row0, BLK_ROWS)],
            )

    return k(h, src, dst, ef)


def _final_body(p_ref, x0_ref, w2_ref, b2_ref, w_ref, b_ref, out_ref):
    agg = p_ref[0] + p_ref[1]
    out = jnp.dot(agg, w2_ref[...], preferred_element_type=jnp.float32) + b2_ref[...]
    out = out + out
    x = _ssp(out)
    x = jnp.dot(x, w_ref[...], preferred_element_type=jnp.float32) + b_ref[...]
    out_ref[...] = x + x0_ref[...]


def _final(partials, node_feats, w2t, b2, wt, b):
    return pl.pallas_call(
        _final_body,
        out_shape=jax.ShapeDtypeStruct((N, H), jnp.float32),
    )(partials, node_feats, w2t, b2, wt, b)


def kernel(node_feats, edge_feats, distances, edge_index,
           mlp_W1, mlp_b1, mlp_W2, mlp_b2,
           lin1_W, lin1_b, lin2_W, lin2_b, lin_W, lin_b):
    src = edge_index[0]
    dst = edge_index[1]
    ef = _edge_mlp(edge_feats, distances[:, None],
                   mlp_W1.T, mlp_b1[None], mlp_W2.T, mlp_b2[None])
    h = _node_lin(node_feats, lin1_W.T, lin1_b[None])
    partials = _sc_msg_agg(h, src, dst, ef)
    return _final(partials, node_feats, lin2_W.T, lin2_b[None],
                  lin_W.T, lin_b[None])
